# baseline (speedup 1.0000x reference)
import jax
import jax.numpy as jnp
from jax.experimental import pallas as pl
from jax.experimental.pallas import tpu as pltpu

_NUM_SEGMENTS = 4096
_N = 6400000
_COLS = 128
_ROWS = _N // _COLS
_BLOCK_ROWS = 10000
_GRID = _ROWS // _BLOCK_ROWS


def _loss_kernel(mu_ref, lv_ref, logit_ref, label_ref, out_ref, acc_ref):
    i = pl.program_id(0)

    @pl.when(i == 0)
    def _init():
        acc_ref[0] = 0.0

    mu = mu_ref[...]
    lv = lv_ref[...]
    acc_ref[0] += jnp.sum(lv - mu * mu - jnp.exp(lv))

    @pl.when(i == _GRID - 1)
    def _finish():
        logits = logit_ref[...]
        labels = label_ref[...].astype(jnp.float32)
        pred = jnp.sum(
            jnp.maximum(logits, 0.0) - logits * labels
            + jnp.log1p(jnp.exp(-jnp.abs(logits)))
        ) / _NUM_SEGMENTS
        total = acc_ref[0] + jnp.float32(_N)
        out_ref[0, 0] = pred + (-0.5) * total / _NUM_SEGMENTS


def kernel(mu, log_var, clf_logits, clf_labels, batch):
    del batch
    mu2 = mu.reshape(_ROWS, _COLS)
    lv2 = log_var.reshape(_ROWS, _COLS)
    logits2 = clf_logits.reshape(32, 128)
    labels2 = clf_labels.reshape(32, 128)
    out = pl.pallas_call(
        _loss_kernel,
        grid=(_GRID,),
        in_specs=[
            pl.BlockSpec((_BLOCK_ROWS, _COLS), lambda i: (i, 0)),
            pl.BlockSpec((_BLOCK_ROWS, _COLS), lambda i: (i, 0)),
            pl.BlockSpec((32, 128), lambda i: (0, 0)),
            pl.BlockSpec((32, 128), lambda i: (0, 0)),
        ],
        out_specs=pl.BlockSpec(memory_space=pltpu.SMEM),
        out_shape=jax.ShapeDtypeStruct((1, 1), jnp.float32),
        scratch_shapes=[pltpu.SMEM((1,), jnp.float32)],
    )(mu2, lv2, logits2, labels2)
    return out[0, 0]
